# baseline (speedup 1.0000x reference)
import functools

import jax
import jax.numpy as jnp
from jax import lax
from jax.experimental import pallas as pl
from jax.experimental.pallas import tpu as pltpu
from jax.experimental.pallas import tpu_sc as plsc

B_TOK = 16384
SEQ = 50
D_MODEL = 64
LANES = 128


@jax.jit
def _embedding_gather(ids_t, weight):
    info = plsc.get_sparse_core_info()
    num_workers = info.num_cores * info.num_subcores
    n_tc = B_TOK // LANES
    tc_per_w = n_tc // num_workers
    mesh = plsc.VectorSubcoreMesh(core_axis_name="c", subcore_axis_name="s")

    @functools.partial(
        pl.kernel,
        mesh=mesh,
        out_type=jax.ShapeDtypeStruct(
            (SEQ, D_MODEL // 8, n_tc, 8, LANES), jnp.float32
        ),
        scratch_types=[
            pltpu.VMEM((SEQ, LANES), jnp.int32),
            pltpu.VMEM((LANES, D_MODEL), jnp.float32),
            pltpu.VMEM((LANES, D_MODEL), jnp.float32),
            pltpu.VMEM((D_MODEL, LANES), jnp.float32),
            pltpu.VMEM((D_MODEL, LANES), jnp.float32),
            pltpu.SemaphoreType.DMA,
            pltpu.SemaphoreType.DMA,
            pltpu.SemaphoreType.DMA,
            pltpu.SemaphoreType.DMA,
            pltpu.SemaphoreType.DMA,
        ],
        compiler_params=pltpu.CompilerParams(
            use_tc_tiling_on_sc=False, needs_layout_passes=False
        ),
    )
    def k(ids_hbm, table_hbm, out_hbm, ids_v, r0, r1, t0, t1, isem,
          g0, g1, w0, w1):
        rbuf = (r0, r1)
        tbuf = (t0, t1)
        gsem = (g0, g1)
        wsem = (w0, w1)
        wid = lax.axis_index("s") * info.num_cores + lax.axis_index("c")
        lane_iota = lax.iota(jnp.int32, 16)
        zeros16 = lane_iota * 0

        def start_ids(tc):
            for s in range(SEQ):
                pltpu.async_copy(
                    ids_hbm.at[s, pl.ds(tc * LANES, LANES)], ids_v.at[s], isem
                )

        def wait_ids(tc):
            for s in range(SEQ):
                pltpu.make_async_copy(
                    ids_hbm.at[s, pl.ds(tc * LANES, LANES)], ids_v.at[s], isem
                ).wait()

        def start_gather(s, p):
            pltpu.async_copy(table_hbm.at[ids_v.at[s]], rbuf[p], gsem[p])

        def wait_gather(s, p):
            pltpu.make_async_copy(
                table_hbm.at[ids_v.at[s]], rbuf[p], gsem[p]
            ).wait()

        def transpose(p):
            rows = rbuf[p]
            dst = tbuf[p]

            def drow(d, carry):
                col = zeros16 + d
                for kk in range(LANES // 16):
                    v = plsc.load_gather(rows, [lane_iota + 16 * kk, col])
                    dst[d, pl.ds(16 * kk, 16)] = v
                return carry

            lax.fori_loop(0, D_MODEL, drow, 0, unroll=8)

        def start_write(s, tc, p):
            for tr in range(D_MODEL // 8):
                pltpu.async_copy(
                    tbuf[p].at[pl.ds(tr * 8, 8)],
                    out_hbm.at[s, tr, tc],
                    wsem[p],
                )

        def wait_write(s, tc, p):
            for tr in range(D_MODEL // 8):
                pltpu.make_async_copy(
                    tbuf[p].at[pl.ds(tr * 8, 8)],
                    out_hbm.at[s, tr, tc],
                    wsem[p],
                ).wait()

        def do_tc(i, carry):
            tc = wid * tc_per_w + i
            start_ids(tc)
            wait_ids(tc)
            start_gather(0, 0)
            start_gather(1, 1)

            def pair(t, carry2):
                s0 = 2 * t
                wait_gather(s0, 0)
                wait_write(s0 - 2, tc, 0)
                transpose(0)
                start_write(s0, tc, 0)
                start_gather(s0 + 2, 0)
                wait_gather(s0 + 1, 1)
                wait_write(s0 - 1, tc, 1)
                transpose(1)
                start_write(s0 + 1, tc, 1)
                start_gather(s0 + 3, 1)
                return carry2

            wait_gather(0, 0)
            transpose(0)
            start_write(0, tc, 0)
            start_gather(2, 0)
            wait_gather(1, 1)
            transpose(1)
            start_write(1, tc, 1)
            start_gather(3, 1)
            lax.fori_loop(1, SEQ // 2 - 1, pair, 0)
            s0 = SEQ - 2
            wait_gather(s0, 0)
            wait_write(s0 - 2, tc, 0)
            transpose(0)
            start_write(s0, tc, 0)
            wait_gather(s0 + 1, 1)
            wait_write(s0 - 1, tc, 1)
            transpose(1)
            start_write(s0 + 1, tc, 1)
            wait_write(s0, tc, 0)
            wait_write(s0 + 1, tc, 1)
            return carry

        lax.fori_loop(0, tc_per_w, do_tc, 0)

    return k(ids_t, weight)


def kernel(token_ids, weight):
    ids_t = jnp.transpose(token_ids)
    p = _embedding_gather(ids_t, weight)
    return jnp.transpose(p, (2, 4, 0, 1, 3)).reshape(B_TOK, SEQ, D_MODEL)
